# baseline (speedup 1.0000x reference)
import functools

import jax
import jax.numpy as jnp
import numpy as np
from jax import lax
from jax.experimental import pallas as pl
from jax.experimental.pallas import tpu as pltpu
from jax.experimental.pallas import tpu_sc as plsc

NC = 2
NS = 16
NW = NC * NS
L = 16
CH = 512
SUB = 128
DPAD = 8

_ACOS_POLY = (1.5707963050, -0.2145988016, 0.0889789874, -0.0501743046,
              0.0308918810, -0.0170881256, 0.0066700901, -0.0012624911)


def _f32(x):
    return jnp.float32(x)


def _rsqrt_nr(u):
    i = plsc.bitcast(u, jnp.int32)
    i = jnp.int32(0x5F3759DF) - lax.shift_right_logical(i, 1)
    y = plsc.bitcast(i, jnp.float32)
    for _ in range(3):
        y = y * (_f32(1.5) - _f32(0.5) * u * y * y)
    return y


def _acos(x):
    ax = jnp.abs(x)
    u = _f32(1.0) - ax
    s = u * _rsqrt_nr(u)
    p = jnp.full((L,), _ACOS_POLY[7], dtype=jnp.float32)
    for c in _ACOS_POLY[6::-1]:
        p = p * ax + _f32(c)
    a = s * p
    return jnp.where(x >= _f32(0.0), a, _f32(np.pi) - a)


def _edge_cos(av, bv, cv, dv, o):
    e0 = bv[o] - av[o]
    e1 = bv[o + 1] - av[o + 1]
    e2 = bv[o + 2] - av[o + 2]
    f0 = cv[o] - av[o]
    f1 = cv[o + 1] - av[o + 1]
    f2 = cv[o + 2] - av[o + 2]
    g0 = dv[o] - av[o]
    g1 = dv[o + 1] - av[o + 1]
    g2 = dv[o + 2] - av[o + 2]
    n10 = e1 * f2 - e2 * f1
    n11 = e2 * f0 - e0 * f2
    n12 = e0 * f1 - e1 * f0
    n20 = e1 * g2 - e2 * g1
    n21 = e2 * g0 - e0 * g2
    n22 = e0 * g1 - e1 * g0
    dot = n10 * n20 + n11 * n21 + n12 * n22
    s1 = n10 * n10 + n11 * n11 + n12 * n12
    s2 = n20 * n20 + n21 * n21 + n22 * n22
    eps = _f32(1e-8)
    den = (s1 * _rsqrt_nr(s1) + eps) * (s2 * _rsqrt_nr(s2) + eps)
    c = dot / den
    return jnp.clip(c, _f32(-1.0 + 1e-6), _f32(1.0 - 1e-6))


@functools.lru_cache(maxsize=None)
def _build_sc(ni, n_vert, n_edge_rows, ee):
    mesh = plsc.VectorSubcoreMesh(core_axis_name="c", subcore_axis_name="s")
    nsub = CH // SUB

    def gathers(table_hbm, idx, rows, sem):
        for v in range(4):
            for s in range(nsub):
                pltpu.async_copy(
                    table_hbm.at[idx.at[v, pl.ds(s * SUB, SUB)]],
                    rows.at[pl.ds(v * CH + s * SUB, SUB), :], sem)

    def drain_gathers(table_hbm, idx, rows, sem):
        for v in range(4):
            for s in range(nsub):
                pltpu.make_async_copy(
                    table_hbm.at[idx.at[v, pl.ds(s * SUB, SUB)]],
                    rows.at[pl.ds(v * CH + s * SUB, SUB), :], sem).wait()

    def clamped_base(chunk):
        return jnp.minimum(chunk * CH, n_edge_rows - CH)

    def build_idx(epb, idx, cbase):
        def group(g, carry):
            pos = lax.iota(jnp.int32, L) + g * L
            off = jnp.zeros((L,), jnp.int32)
            gid = pos + cbase
            for b in range(1, 4):
                off = off + jnp.where(gid >= b * ee, jnp.int32(n_vert),
                                      jnp.int32(0))
            for v in range(4):
                col = plsc.load_gather(
                    epb, [pos, jnp.full((L,), v, dtype=jnp.int32)])
                idx[v, pl.ds(g * L, L)] = col + off
            return carry

        lax.fori_loop(0, CH // L, group, 0)

    def compute(rows, dneg, acc):
        def group(g, acc):
            av, bv, cv, dv = [
                [plsc.load_gather(
                    rows,
                    [lax.iota(jnp.int32, L) + (g * L + v * CH),
                     jnp.full((L,), c, dtype=jnp.int32)])
                 for c in range(6)]
                for v in range(4)]
            c1 = _edge_cos(av, bv, cv, dv, 0)
            c2 = _edge_cos(av, bv, cv, dv, 3)
            t = _acos(c2) - _acos(c1)
            pos = lax.iota(jnp.int32, L) + (g * L) + dneg
            t = jnp.where(pos >= 0, t, _f32(0.0))
            return acc + t * t

        return lax.fori_loop(0, CH // L, group, acc)

    def body(ep_hbm, table_hbm, out_hbm,
             epb_a, epb_b, idx_a, idx_b, rows_a, rows_b, accv,
             sem_a, sem_b, sem_ea, sem_eb):
        wid = lax.axis_index("s") * NC + lax.axis_index("c")
        w_chunk = wid * ni

        def load_ep(chunk, epb, sem):
            pltpu.async_copy(
                ep_hbm.at[pl.ds(clamped_base(chunk), CH), :], epb, sem)

        def wait_ep(chunk, epb, sem):
            pltpu.make_async_copy(
                ep_hbm.at[pl.ds(clamped_base(chunk), CH), :], epb, sem).wait()

        pltpu.sync_copy(
            ep_hbm.at[pl.ds(clamped_base(w_chunk), CH), :], epb_a)
        build_idx(epb_a, idx_a, clamped_base(w_chunk))
        gathers(table_hbm, idx_a, rows_a, sem_a)
        load_ep(w_chunk + 1, epb_b, sem_eb)

        def pair(j, acc):
            c0 = w_chunk + 2 * j
            wait_ep(c0 + 1, epb_b, sem_eb)
            build_idx(epb_b, idx_b, clamped_base(c0 + 1))
            gathers(table_hbm, idx_b, rows_b, sem_b)

            @pl.when(2 * j + 2 < ni)
            def _prefetch_a():
                load_ep(c0 + 2, epb_a, sem_ea)

            drain_gathers(table_hbm, idx_a, rows_a, sem_a)
            acc = compute(rows_a, clamped_base(c0) - c0 * CH, acc)

            @pl.when(2 * j + 2 < ni)
            def _launch_a():
                wait_ep(c0 + 2, epb_a, sem_ea)
                build_idx(epb_a, idx_a, clamped_base(c0 + 2))
                gathers(table_hbm, idx_a, rows_a, sem_a)

            drain_gathers(table_hbm, idx_b, rows_b, sem_b)

            @pl.when(2 * j + 3 < ni)
            def _prefetch_b():
                load_ep(c0 + 3, epb_b, sem_eb)

            return compute(rows_b, clamped_base(c0 + 1) - (c0 + 1) * CH, acc)

        acc = lax.fori_loop(0, ni // 2, pair, jnp.zeros((L,), jnp.float32))
        accv[...] = acc
        pltpu.sync_copy(accv, out_hbm.at[wid])

    return pl.kernel(
        body,
        out_type=jax.ShapeDtypeStruct((NW, L), jnp.float32),
        mesh=mesh,
        compiler_params=pltpu.CompilerParams(
            needs_layout_passes=False, use_tc_tiling_on_sc=False),
        scratch_types=[
            pltpu.VMEM((CH, 4), jnp.int32),
            pltpu.VMEM((CH, 4), jnp.int32),
            pltpu.VMEM((4, CH), jnp.int32),
            pltpu.VMEM((4, CH), jnp.int32),
            pltpu.VMEM((4 * CH, DPAD), jnp.float32),
            pltpu.VMEM((4 * CH, DPAD), jnp.float32),
            pltpu.VMEM((L,), jnp.float32),
            pltpu.SemaphoreType.DMA,
            pltpu.SemaphoreType.DMA,
            pltpu.SemaphoreType.DMA,
            pltpu.SemaphoreType.DMA,
        ],
    )


@functools.lru_cache(maxsize=None)
def _build_tc(denom):
    def body(part_ref, out_ref):
        out_ref[0, 0] = jnp.sum(part_ref[...]) * _f32(1.0 / denom)

    return pl.pallas_call(
        body,
        out_shape=jax.ShapeDtypeStruct((1, 1), jnp.float32),
        out_specs=pl.BlockSpec(memory_space=pltpu.SMEM),
    )


def kernel(vert1, vert2, edge_points):
    B, N, _ = vert1.shape
    E = edge_points.shape[1]
    BE = B * E
    nch = -(-BE // CH)
    ni = -(-nch // NW)
    ni += ni % 2

    table = jnp.concatenate(
        [vert1, vert2, jnp.zeros((B, N, DPAD - 6), jnp.float32)],
        axis=-1).reshape(B * N, DPAD)
    ep = edge_points.astype(jnp.int32).reshape(BE, 4)

    part = _build_sc(ni, N, BE, E)(ep, table)
    res = _build_tc(float(BE))(part)
    return res[0, 0]
